# baseline (speedup 1.0000x reference)
import math

import jax
import jax.numpy as jnp
from jax.experimental import pallas as pl

NS = 32
NV = 8
NB = 8
L = 4
MAX_RADIUS = 5.0
NUM_NEIGHBORS = 16.0


def _bessel(x, nb, end):
    n = jnp.arange(1, nb + 1, dtype=jnp.float32) * math.pi
    safe = jnp.clip(x, 1e-6, None)
    out = jnp.sqrt(2.0 / end) * jnp.sin(n[None, :] * (safe[:, None] / end)) / safe[:, None]
    return out * (nb ** 0.5)


def _cutoff(x):
    u = 2.0 * (x - 1.0)
    y = (1.0 - jnp.cos(math.pi * u)) / 2.0
    y = jnp.where(u > 0, 0.0, y)
    y = jnp.where(u < -2, 1.0, y)
    return y


def _copy_kernel(x_ref, o_ref):
    o_ref[...] = x_ref[...]


def kernel(x, batch, node_attr, edge_src, edge_dst, embed, W_up, W_proj, Wr1, Wr2, B, C, Wemb, Wg, Wsi_s, Wsi_v, h, mix):
    atom = jnp.min(node_attr, axis=-1)
    emb = embed[atom]
    v = x[:, None, :] * W_up[None, :, None]
    s = jnp.zeros((x.shape[0], NS), dtype=x.dtype)
    s_old, v_old = s, v
    edge_vec = x[edge_src] - x[edge_dst]
    edge_len = jnp.linalg.norm(edge_vec, axis=1)
    edge_feat = _bessel(edge_len, NB, MAX_RADIUS)
    sh = math.sqrt(3.0) * edge_vec / jnp.clip(edge_len, 1e-6, None)[:, None]
    edge_attr = _cutoff(edge_len / MAX_RADIUS)[:, None] * sh
    inv = 1.0 / math.sqrt(NUM_NEIGHBORS)
    for i in range(L):
        dt = jnp.clip(h[i] ** 2, 1e-4, 0.1)
        w = jax.nn.silu(edge_feat @ Wr1[i]) @ Wr2[i]
        w1 = w[:, :NS]
        w2 = w[:, NS:NS + NV]
        w3 = w[:, NS + NV:2 * NS + NV]
        w4 = w[:, 2 * NS + NV:]
        src_s = s[edge_src]
        src_v = v[edge_src]
        dotv = jnp.einsum('evc,ec->ev', src_v, edge_attr)
        m_s = src_s * w1 + (dotv * w2) @ B[i]
        m_v = ((src_s * w3) @ C[i])[:, :, None] * edge_attr[:, None, :] + src_v * w4[:, :, None]
        agg_s = jnp.zeros_like(s).at[edge_dst].add(m_s) * inv + emb @ Wemb[i]
        agg_v = jnp.zeros_like(v).at[edge_dst].add(m_v) * inv
        g = jax.nn.sigmoid(agg_s @ Wg[i])
        cs = jax.nn.silu(agg_s)
        cv = agg_v * g[:, :, None]
        si_s = s @ Wsi_s[i]
        si_v = jnp.einsum('nvc,vw->nwc', v, Wsi_v[i])
        m = jnp.minimum(mix[i] ** 2, 1.0)
        y_s = m * cs + (1.0 - m) * si_s
        y_v = m * cv + (1.0 - m) * si_v
        tmp_s, tmp_v = s, v
        s = 2.0 * s - s_old + dt * y_s
        v = 2.0 * v - v_old + dt * y_v
        s_old, v_old = tmp_s, tmp_v
    x_out = jnp.einsum('nvc,v->nc', v, W_proj)
    x_out = pl.pallas_call(
        _copy_kernel,
        grid=(100,),
        in_specs=[pl.BlockSpec((1000, 3), lambda i: (i, 0))],
        out_specs=pl.BlockSpec((1000, 3), lambda i: (i, 0)),
        out_shape=jax.ShapeDtypeStruct(x_out.shape, x_out.dtype),
    )(x_out)
    return x_out


# speedup vs baseline: 11.3447x; 11.3447x over previous
import functools
import math

import numpy as np
import jax
import jax.numpy as jnp
from jax import lax
from jax.experimental import pallas as pl
from jax.experimental.pallas import tpu as pltpu
from jax.experimental.pallas import tpu_sc as plsc

N = 100000
E = 1600000
NS = 32
NV = 8
NB = 8
L = 4
MAX_RADIUS = 5.0
INV_NEIGH = 1.0 / math.sqrt(16.0)

ROWS = E // 128
ROWS_P = 12512
EP = ROWS_P * 128
BE = 512
BN = 2000

HIGH = jax.lax.Precision.HIGHEST


def _dot(a, b):
    return jnp.dot(a, b, precision=HIGH, preferred_element_type=jnp.float32)


def _iota2(shape, dim):
    return lax.broadcasted_iota(jnp.int32, shape, dim)


def _sel_s():
    return (_iota2((64, 32), 0) == _iota2((64, 32), 1)).astype(jnp.float32)


def _sel_v():
    return (_iota2((64, 24), 0) - 32 == _iota2((64, 24), 1)).astype(jnp.float32)


def _g_mat():
    return (_iota2((24, 8), 0) // 3 == _iota2((24, 8), 1)).astype(jnp.float32)


def _gt_mat():
    return (_iota2((8, 24), 1) // 3 == _iota2((8, 24), 0)).astype(jnp.float32)


def _t_mat():
    i0 = _iota2((8, 24), 0)
    return ((_iota2((8, 24), 1) % 3 == i0) & (i0 < 3)).astype(jnp.float32)


def _init_body(xp_ref, q_ref, st_ref):
    v0 = _dot(xp_ref[...], q_ref[...])
    z32 = jnp.zeros((xp_ref.shape[0], 32), jnp.float32)
    z8 = jnp.zeros((xp_ref.shape[0], 8), jnp.float32)
    st_ref[...] = jnp.concatenate([z32, v0, z8], axis=1)


def _efeat_body(xs_ref, xd_ref, ef_ref, at_ref):
    vec = xs_ref[...] - xd_ref[...]
    l2 = jnp.sum(vec * vec, axis=1, keepdims=True)
    ln = jnp.sqrt(l2)
    safe = jnp.maximum(ln, 1e-6)
    n_pi = (lax.broadcasted_iota(jnp.int32, vec.shape, 1).astype(jnp.float32)
            + 1.0) * math.pi
    ef = math.sqrt(2.0 / MAX_RADIUS) * jnp.sin(n_pi * (safe / MAX_RADIUS)) / safe
    ef_ref[...] = ef * (NB ** 0.5)
    u = 2.0 * (ln / MAX_RADIUS - 1.0)
    y = (1.0 - jnp.cos(math.pi * u)) / 2.0
    y = jnp.where(u > 0, 0.0, y)
    y = jnp.where(u < -2, 1.0, y)
    at_ref[...] = y * math.sqrt(3.0) * vec / safe


def _edge_body(ef_ref, at_ref, gst_ref, wr1_ref, w1m_ref, w2m_ref, w3m_ref,
               w4m_ref, b_ref, c_ref, m_ref):
    ef = ef_ref[...]
    attr = at_ref[...]
    gst = gst_ref[...]
    hmid = jax.nn.silu(_dot(ef, wr1_ref[...]))
    w1 = _dot(hmid, w1m_ref[...])
    w2 = _dot(hmid, w2m_ref[...])
    w3 = _dot(hmid, w3m_ref[...])
    w4 = _dot(hmid, w4m_ref[...])
    gs = _dot(gst, _sel_s())
    gv = _dot(gst, _sel_v())
    a24 = _dot(attr, _t_mat())
    dotv = _dot(gv * a24, _g_mat())
    m_s = gs * w1 + _dot(dotv * w2, b_ref[...])
    u = _dot(gs * w3, c_ref[...])
    m_v = _dot(u, _gt_mat()) * a24 + gv * _dot(w4, _gt_mat())
    z8 = jnp.zeros((ef.shape[0], 8), jnp.float32)
    m_ref[...] = jnp.concatenate([m_s, m_v, z8], axis=1)


def _node_body(agg_ref, na_ref, st_ref, sto_ref, embw_ref, wg_ref, wsis_ref,
               k24_ref, par_ref, out_ref):
    agg = agg_ref[...]
    st = st_ref[...]
    sto = sto_ref[...]
    dt = par_ref[0:1, 0:1]
    mm = par_ref[0:1, 1:2]
    sel_s = _sel_s()
    sel_v = _sel_v()
    oh = (na_ref[...] == lax.broadcasted_iota(jnp.int32, (agg.shape[0], 16), 1)
          ).astype(jnp.float32)
    agg_s = _dot(agg, sel_s) * INV_NEIGH + _dot(oh, embw_ref[...])
    agg_v = _dot(agg, sel_v) * INV_NEIGH
    s = _dot(st, sel_s)
    v = _dot(st, sel_v)
    s_o = _dot(sto, sel_s)
    v_o = _dot(sto, sel_v)
    g8 = jax.nn.sigmoid(_dot(agg_s, wg_ref[...]))
    cs = jax.nn.silu(agg_s)
    cv = agg_v * _dot(g8, _gt_mat())
    si_s = _dot(s, wsis_ref[...])
    si_v = _dot(v, k24_ref[...])
    y_s = mm * cs + (1.0 - mm) * si_s
    y_v = mm * cv + (1.0 - mm) * si_v
    s_n = 2.0 * s - s_o + dt * y_s
    v_n = 2.0 * v - v_o + dt * y_v
    z8 = jnp.zeros((agg.shape[0], 8), jnp.float32)
    out_ref[...] = jnp.concatenate([s_n, v_n, z8], axis=1)


def _proj_body(st_ref, p_ref, out_ref):
    out_ref[...] = _dot(st_ref[...], p_ref[...])


def _full(shape):
    return pl.BlockSpec(shape, lambda i: tuple(0 for _ in shape))


_GE = EP // BE
_GN = N // BN

_init_call = pl.pallas_call(
    _init_body,
    grid=(_GN,),
    in_specs=[pl.BlockSpec((BN, 8), lambda i: (i, 0)), _full((8, 24))],
    out_specs=pl.BlockSpec((BN, 64), lambda i: (i, 0)),
    out_shape=jax.ShapeDtypeStruct((N, 64), jnp.float32),
)

_efeat_call = pl.pallas_call(
    _efeat_body,
    grid=(_GE,),
    in_specs=[pl.BlockSpec((BE, 8), lambda i: (i, 0))] * 2,
    out_specs=[pl.BlockSpec((BE, 8), lambda i: (i, 0))] * 2,
    out_shape=[jax.ShapeDtypeStruct((EP, 8), jnp.float32)] * 2,
)

_edge_call = pl.pallas_call(
    _edge_body,
    grid=(_GE,),
    in_specs=[
        pl.BlockSpec((BE, 8), lambda i: (i, 0)),
        pl.BlockSpec((BE, 8), lambda i: (i, 0)),
        pl.BlockSpec((BE, 64), lambda i: (i, 0)),
        _full((8, 16)), _full((16, 32)), _full((16, 8)), _full((16, 32)),
        _full((16, 8)), _full((8, 32)), _full((32, 8)),
    ],
    out_specs=pl.BlockSpec((BE, 64), lambda i: (i, 0)),
    out_shape=jax.ShapeDtypeStruct((EP, 64), jnp.float32),
)

_node_call = pl.pallas_call(
    _node_body,
    grid=(_GN,),
    in_specs=[
        pl.BlockSpec((BN, 64), lambda i: (i, 0)),
        pl.BlockSpec((BN, 1), lambda i: (i, 0)),
        pl.BlockSpec((BN, 64), lambda i: (i, 0)),
        pl.BlockSpec((BN, 64), lambda i: (i, 0)),
        _full((16, 32)), _full((32, 8)), _full((32, 32)), _full((24, 24)),
        _full((1, 8)),
    ],
    out_specs=pl.BlockSpec((BN, 64), lambda i: (i, 0)),
    out_shape=jax.ShapeDtypeStruct((N, 64), jnp.float32),
)

_proj_call = pl.pallas_call(
    _proj_body,
    grid=(_GN,),
    in_specs=[pl.BlockSpec((BN, 64), lambda i: (i, 0)), _full((64, 8))],
    out_specs=pl.BlockSpec((BN, 8), lambda i: (i, 0)),
    out_shape=jax.ShapeDtypeStruct((N, 8), jnp.float32),
)


_mesh = plsc.VectorSubcoreMesh(core_axis_name="c", subcore_axis_name="s")

_ROWS_PER_W32 = ROWS_P // 32
_ROWS_PER_W16 = ROWS_P // 16
_NPT = N // 16


@functools.partial(
    pl.kernel,
    compiler_params=pltpu.CompilerParams(use_tc_tiling_on_sc=False),
    out_type=(jax.ShapeDtypeStruct((EP, 8), jnp.float32),
              jax.ShapeDtypeStruct((EP, 8), jnp.float32)),
    mesh=_mesh,
    scratch_types=[
        pltpu.VMEM((128,), jnp.int32),
        pltpu.VMEM((128,), jnp.int32),
        pltpu.VMEM((128, 8), jnp.float32),
        pltpu.VMEM((128, 8), jnp.float32),
        pltpu.SemaphoreType.DMA,
        pltpu.SemaphoreType.DMA,
    ],
)
def _geom_gather(xp_hbm, src2d, dst2d, xs_out, xd_out,
                 idxs, idxd, bufs, bufd, sems, semd):
    wid = lax.axis_index("s") * 2 + lax.axis_index("c")
    row0 = wid * _ROWS_PER_W32

    def body(j, _):
        r = row0 + j
        pltpu.sync_copy(src2d.at[r], idxs)
        pltpu.sync_copy(dst2d.at[r], idxd)
        a = pltpu.async_copy(xp_hbm.at[idxs], bufs, sems)
        b = pltpu.async_copy(xp_hbm.at[idxd], bufd, semd)
        a.wait()
        b.wait()
        pltpu.sync_copy(bufs, xs_out.at[pl.ds(r * 128, 128), :])
        pltpu.sync_copy(bufd, xd_out.at[pl.ds(r * 128, 128), :])
        return _

    lax.fori_loop(0, _ROWS_PER_W32, body, None)


@functools.partial(
    pl.kernel,
    compiler_params=pltpu.CompilerParams(use_tc_tiling_on_sc=False),
    out_type=jax.ShapeDtypeStruct((EP, 64), jnp.float32),
    mesh=_mesh,
    scratch_types=[
        pltpu.VMEM((128,), jnp.int32),
        pltpu.VMEM((128, 64), jnp.float32),
        pltpu.SemaphoreType.DMA,
    ],
)
def _state_gather(st_hbm, src2d, gst_out, idxb, buf, sem):
    wid = lax.axis_index("s") * 2 + lax.axis_index("c")
    row0 = wid * _ROWS_PER_W32

    def body(j, _):
        r = row0 + j
        pltpu.sync_copy(src2d.at[r], idxb)
        pltpu.async_copy(st_hbm.at[idxb], buf, sem).wait()
        pltpu.sync_copy(buf, gst_out.at[pl.ds(r * 128, 128), :])
        return _

    lax.fori_loop(0, _ROWS_PER_W32, body, None)


@functools.partial(
    pl.kernel,
    compiler_params=pltpu.CompilerParams(use_tc_tiling_on_sc=False),
    out_type=jax.ShapeDtypeStruct((N, 64), jnp.float32),
    mesh=_mesh,
    scratch_types=[
        pltpu.VMEM((17, 128), jnp.int32),
        pltpu.VMEM((17 * 128, 8), jnp.float32),
        pltpu.VMEM_SHARED((N, 8), jnp.float32),
    ],
)
def _scatter_add(m64, dst2d, zeros16, agg_out, idxb, rowb, acc):
    core = lax.axis_index("c")
    sid = lax.axis_index("s")
    row0 = sid * _ROWS_PER_W16

    for g in range(7):
        @pl.when(core == g // 4)
        def _group(g=g):
            c0 = 8 * g
            pltpu.sync_copy(zeros16.at[pl.ds(sid * _NPT, _NPT), 0:8],
                            acc.at[pl.ds(sid * _NPT, _NPT), :])
            plsc.subcore_barrier()

            def body(b, _):
                rb = row0 + b * 17
                pltpu.sync_copy(dst2d.at[pl.ds(rb, 17), :], idxb)
                pltpu.sync_copy(m64.at[pl.ds(rb * 128, 17 * 128),
                                       pl.ds(c0, 8)], rowb)
                for j in range(17):
                    @pl.when(rb + j < ROWS)
                    def _add(j=j):
                        pltpu.sync_copy(rowb.at[pl.ds(j * 128, 128), :],
                                        acc.at[idxb.at[j]], add=True)
                return _

            lax.fori_loop(0, _ROWS_PER_W16 // 17, body, None)
            plsc.subcore_barrier()
            pltpu.sync_copy(acc.at[pl.ds(sid * _NPT, _NPT), :],
                            agg_out.at[pl.ds(sid * _NPT, _NPT), pl.ds(c0, 8)])
            if g == 6:
                pltpu.sync_copy(zeros16.at[pl.ds(sid * _NPT, _NPT), 0:8],
                                agg_out.at[pl.ds(sid * _NPT, _NPT),
                                           pl.ds(56, 8)])
            plsc.subcore_barrier()

    return


def kernel(x, batch, node_attr, edge_src, edge_dst, embed, W_up, W_proj, Wr1,
           Wr2, B, C, Wemb, Wg, Wsi_s, Wsi_v, h, mix):
    f32 = jnp.float32
    xp = jnp.pad(x, ((0, 0), (0, 5)))
    pad_e = EP - E
    src2d = jnp.concatenate(
        [edge_src, jnp.zeros((pad_e,), jnp.int32)]).reshape(ROWS_P, 128)
    dst2d = jnp.concatenate(
        [edge_dst, jnp.zeros((pad_e,), jnp.int32)]).reshape(ROWS_P, 128)
    zeros16 = jnp.zeros((N, 16), f32)
    eye3 = jnp.eye(3, dtype=f32)

    qup = jnp.zeros((8, 24), f32).at[0:3, :].set(
        jnp.kron(W_up[None, :], eye3))
    embed_pad = jnp.pad(embed, ((0, 6), (0, 0)))
    pfull = jnp.zeros((64, 8), f32).at[32:56, 0:3].set(
        jnp.kron(W_proj[:, None], eye3))

    st = _init_call(xp, qup)
    st_old = st
    xs, xd = _geom_gather(xp, src2d, dst2d)
    ef, attr = _efeat_call(xs, xd)

    z = jnp.float32(0.0)
    for i in range(L):
        dt = jnp.clip(h[i] ** 2, 1e-4, 0.1).astype(f32)
        mm = jnp.minimum(mix[i] ** 2, 1.0).astype(f32)
        par = jnp.stack([dt, mm, z, z, z, z, z, z])[None, :]
        embw = jnp.dot(embed_pad, Wemb[i], precision=HIGH)
        k24 = jnp.kron(Wsi_v[i], eye3)
        w1m = Wr2[i][:, :NS]
        w2m = Wr2[i][:, NS:NS + NV]
        w3m = Wr2[i][:, NS + NV:2 * NS + NV]
        w4m = Wr2[i][:, 2 * NS + NV:]

        gst = _state_gather(st, src2d)
        m64 = _edge_call(ef, attr, gst, Wr1[i], w1m, w2m, w3m, w4m, B[i], C[i])
        agg = _scatter_add(m64, dst2d, zeros16)
        st_new = _node_call(agg, node_attr, st, st_old, embw, Wg[i],
                            Wsi_s[i], k24, par)
        st_old, st = st, st_new

    out8 = _proj_call(st, pfull)
    return out8[:, :3]


# speedup vs baseline: 12.2137x; 1.0766x over previous
import functools
import math

import numpy as np
import jax
import jax.numpy as jnp
from jax import lax
from jax.experimental import pallas as pl
from jax.experimental.pallas import tpu as pltpu
from jax.experimental.pallas import tpu_sc as plsc

N = 100000
E = 1600000
NS = 32
NV = 8
NB = 8
L = 4
MAX_RADIUS = 5.0
INV_NEIGH = 1.0 / math.sqrt(16.0)

ROWS = E // 128
ROWS_P = 12512
EP = ROWS_P * 128
BE = 2048
BN = 4000

HIGH = jax.lax.Precision.HIGHEST


def _dot(a, b):
    return jnp.dot(a, b, precision=HIGH, preferred_element_type=jnp.float32)


def _iota2(shape, dim):
    return lax.broadcasted_iota(jnp.int32, shape, dim)


def _sel_s():
    return (_iota2((64, 32), 0) == _iota2((64, 32), 1)).astype(jnp.float32)


def _sel_v():
    return (_iota2((64, 24), 0) - 32 == _iota2((64, 24), 1)).astype(jnp.float32)


def _g_mat():
    return (_iota2((24, 8), 0) // 3 == _iota2((24, 8), 1)).astype(jnp.float32)


def _gt_mat():
    return (_iota2((8, 24), 1) // 3 == _iota2((8, 24), 0)).astype(jnp.float32)


def _t_mat():
    i0 = _iota2((8, 24), 0)
    return ((_iota2((8, 24), 1) % 3 == i0) & (i0 < 3)).astype(jnp.float32)


def _init_body(xp_ref, q_ref, st_ref):
    v0 = _dot(xp_ref[...], q_ref[...])
    z32 = jnp.zeros((xp_ref.shape[0], 32), jnp.float32)
    z8 = jnp.zeros((xp_ref.shape[0], 8), jnp.float32)
    st_ref[...] = jnp.concatenate([z32, v0, z8], axis=1)


def _efeat_body(xs_ref, xd_ref, ef_ref, at_ref):
    vec = xs_ref[...] - xd_ref[...]
    l2 = jnp.sum(vec * vec, axis=1, keepdims=True)
    ln = jnp.sqrt(l2)
    safe = jnp.maximum(ln, 1e-6)
    n_pi = (lax.broadcasted_iota(jnp.int32, vec.shape, 1).astype(jnp.float32)
            + 1.0) * math.pi
    ef = math.sqrt(2.0 / MAX_RADIUS) * jnp.sin(n_pi * (safe / MAX_RADIUS)) / safe
    ef_ref[...] = ef * (NB ** 0.5)
    u = 2.0 * (ln / MAX_RADIUS - 1.0)
    y = (1.0 - jnp.cos(math.pi * u)) / 2.0
    y = jnp.where(u > 0, 0.0, y)
    y = jnp.where(u < -2, 1.0, y)
    at_ref[...] = y * math.sqrt(3.0) * vec / safe


def _edge_body(ef_ref, at_ref, gst_ref, wr1_ref, w1m_ref, w2m_ref, w3m_ref,
               w4m_ref, b_ref, c_ref, m_ref):
    ef = ef_ref[...]
    attr = at_ref[...]
    gst = gst_ref[...]
    hmid = jax.nn.silu(_dot(ef, wr1_ref[...]))
    w1 = _dot(hmid, w1m_ref[...])
    w2 = _dot(hmid, w2m_ref[...])
    w3 = _dot(hmid, w3m_ref[...])
    w4 = _dot(hmid, w4m_ref[...])
    gs = _dot(gst, _sel_s())
    gv = _dot(gst, _sel_v())
    a24 = _dot(attr, _t_mat())
    dotv = _dot(gv * a24, _g_mat())
    m_s = gs * w1 + _dot(dotv * w2, b_ref[...])
    u = _dot(gs * w3, c_ref[...])
    m_v = _dot(u, _gt_mat()) * a24 + gv * _dot(w4, _gt_mat())
    z8 = jnp.zeros((ef.shape[0], 8), jnp.float32)
    m_ref[...] = jnp.concatenate([m_s, m_v, z8], axis=1)


def _node_body(agg_ref, na_ref, st_ref, sto_ref, embw_ref, wg_ref, wsis_ref,
               k24_ref, par_ref, out_ref):
    agg = agg_ref[...]
    st = st_ref[...]
    sto = sto_ref[...]
    dt = par_ref[0:1, 0:1]
    mm = par_ref[0:1, 1:2]
    sel_s = _sel_s()
    sel_v = _sel_v()
    oh = (na_ref[...] == lax.broadcasted_iota(jnp.int32, (agg.shape[0], 16), 1)
          ).astype(jnp.float32)
    agg_s = _dot(agg, sel_s) * INV_NEIGH + _dot(oh, embw_ref[...])
    agg_v = _dot(agg, sel_v) * INV_NEIGH
    s = _dot(st, sel_s)
    v = _dot(st, sel_v)
    s_o = _dot(sto, sel_s)
    v_o = _dot(sto, sel_v)
    g8 = jax.nn.sigmoid(_dot(agg_s, wg_ref[...]))
    cs = jax.nn.silu(agg_s)
    cv = agg_v * _dot(g8, _gt_mat())
    si_s = _dot(s, wsis_ref[...])
    si_v = _dot(v, k24_ref[...])
    y_s = mm * cs + (1.0 - mm) * si_s
    y_v = mm * cv + (1.0 - mm) * si_v
    s_n = 2.0 * s - s_o + dt * y_s
    v_n = 2.0 * v - v_o + dt * y_v
    z8 = jnp.zeros((agg.shape[0], 8), jnp.float32)
    out_ref[...] = jnp.concatenate([s_n, v_n, z8], axis=1)


def _proj_body(st_ref, p_ref, out_ref):
    out_ref[...] = _dot(st_ref[...], p_ref[...])


def _full(shape):
    return pl.BlockSpec(shape, lambda i: tuple(0 for _ in shape))


_GE = EP // BE
_GN = N // BN

_init_call = pl.pallas_call(
    _init_body,
    grid=(_GN,),
    in_specs=[pl.BlockSpec((BN, 8), lambda i: (i, 0)), _full((8, 24))],
    out_specs=pl.BlockSpec((BN, 64), lambda i: (i, 0)),
    out_shape=jax.ShapeDtypeStruct((N, 64), jnp.float32),
)

_efeat_call = pl.pallas_call(
    _efeat_body,
    grid=(_GE,),
    in_specs=[pl.BlockSpec((BE, 8), lambda i: (i, 0))] * 2,
    out_specs=[pl.BlockSpec((BE, 8), lambda i: (i, 0))] * 2,
    out_shape=[jax.ShapeDtypeStruct((EP, 8), jnp.float32)] * 2,
)

_edge_call = pl.pallas_call(
    _edge_body,
    grid=(_GE,),
    in_specs=[
        pl.BlockSpec((BE, 8), lambda i: (i, 0)),
        pl.BlockSpec((BE, 8), lambda i: (i, 0)),
        pl.BlockSpec((BE, 64), lambda i: (i, 0)),
        _full((8, 16)), _full((16, 32)), _full((16, 8)), _full((16, 32)),
        _full((16, 8)), _full((8, 32)), _full((32, 8)),
    ],
    out_specs=pl.BlockSpec((BE, 64), lambda i: (i, 0)),
    out_shape=jax.ShapeDtypeStruct((EP, 64), jnp.float32),
)

_node_call = pl.pallas_call(
    _node_body,
    grid=(_GN,),
    in_specs=[
        pl.BlockSpec((BN, 64), lambda i: (i, 0)),
        pl.BlockSpec((BN, 1), lambda i: (i, 0)),
        pl.BlockSpec((BN, 64), lambda i: (i, 0)),
        pl.BlockSpec((BN, 64), lambda i: (i, 0)),
        _full((16, 32)), _full((32, 8)), _full((32, 32)), _full((24, 24)),
        _full((1, 8)),
    ],
    out_specs=pl.BlockSpec((BN, 64), lambda i: (i, 0)),
    out_shape=jax.ShapeDtypeStruct((N, 64), jnp.float32),
)

_proj_call = pl.pallas_call(
    _proj_body,
    grid=(_GN,),
    in_specs=[pl.BlockSpec((BN, 64), lambda i: (i, 0)), _full((64, 8))],
    out_specs=pl.BlockSpec((BN, 8), lambda i: (i, 0)),
    out_shape=jax.ShapeDtypeStruct((N, 8), jnp.float32),
)


_mesh = plsc.VectorSubcoreMesh(core_axis_name="c", subcore_axis_name="s")

_ROWS_PER_W32 = ROWS_P // 32
_ROWS_PER_W16 = ROWS_P // 16
_NPT = N // 16


@functools.partial(
    pl.kernel,
    compiler_params=pltpu.CompilerParams(use_tc_tiling_on_sc=False),
    out_type=(jax.ShapeDtypeStruct((EP, 8), jnp.float32),
              jax.ShapeDtypeStruct((EP, 8), jnp.float32)),
    mesh=_mesh,
    scratch_types=[
        pltpu.VMEM((128,), jnp.int32),
        pltpu.VMEM((128,), jnp.int32),
        pltpu.VMEM((128, 8), jnp.float32),
        pltpu.VMEM((128, 8), jnp.float32),
        pltpu.SemaphoreType.DMA,
        pltpu.SemaphoreType.DMA,
    ],
)
def _geom_gather(xp_hbm, src2d, dst2d, xs_out, xd_out,
                 idxs, idxd, bufs, bufd, sems, semd):
    wid = lax.axis_index("s") * 2 + lax.axis_index("c")
    row0 = wid * _ROWS_PER_W32

    def body(j, _):
        r = row0 + j
        pltpu.sync_copy(src2d.at[r], idxs)
        pltpu.sync_copy(dst2d.at[r], idxd)
        a = pltpu.async_copy(xp_hbm.at[idxs], bufs, sems)
        b = pltpu.async_copy(xp_hbm.at[idxd], bufd, semd)
        a.wait()
        b.wait()
        pltpu.sync_copy(bufs, xs_out.at[pl.ds(r * 128, 128), :])
        pltpu.sync_copy(bufd, xd_out.at[pl.ds(r * 128, 128), :])
        return _

    lax.fori_loop(0, _ROWS_PER_W32, body, None)


@functools.partial(
    pl.kernel,
    compiler_params=pltpu.CompilerParams(use_tc_tiling_on_sc=False),
    out_type=jax.ShapeDtypeStruct((EP, 64), jnp.float32),
    mesh=_mesh,
    scratch_types=[
        pltpu.VMEM((128,), jnp.int32),
        pltpu.VMEM((128, 64), jnp.float32),
        pltpu.SemaphoreType.DMA,
    ],
)
def _state_gather(st_hbm, src2d, gst_out, idxb, buf, sem):
    wid = lax.axis_index("s") * 2 + lax.axis_index("c")
    row0 = wid * _ROWS_PER_W32

    def body(j, _):
        r = row0 + j
        pltpu.sync_copy(src2d.at[r], idxb)
        pltpu.async_copy(st_hbm.at[idxb], buf, sem).wait()
        pltpu.sync_copy(buf, gst_out.at[pl.ds(r * 128, 128), :])
        return _

    lax.fori_loop(0, _ROWS_PER_W32, body, None)


@functools.partial(
    pl.kernel,
    compiler_params=pltpu.CompilerParams(use_tc_tiling_on_sc=False),
    out_type=jax.ShapeDtypeStruct((N, 64), jnp.float32),
    mesh=_mesh,
    scratch_types=[
        pltpu.VMEM((17, 128), jnp.int32),
        pltpu.VMEM((17 * 128, 8), jnp.float32),
        pltpu.VMEM_SHARED((N, 8), jnp.float32),
    ],
)
def _scatter_add(m64, dst2d, zeros16, agg_out, idxb, rowb, acc):
    core = lax.axis_index("c")
    sid = lax.axis_index("s")
    row0 = sid * _ROWS_PER_W16

    for g in range(7):
        @pl.when(core == g // 4)
        def _group(g=g):
            c0 = 8 * g
            pltpu.sync_copy(zeros16.at[pl.ds(sid * _NPT, _NPT), 0:8],
                            acc.at[pl.ds(sid * _NPT, _NPT), :])
            plsc.subcore_barrier()

            def body(b, _):
                rb = row0 + b * 17
                pltpu.sync_copy(dst2d.at[pl.ds(rb, 17), :], idxb)
                pltpu.sync_copy(m64.at[pl.ds(rb * 128, 17 * 128),
                                       pl.ds(c0, 8)], rowb)
                for j in range(17):
                    @pl.when(rb + j < ROWS)
                    def _add(j=j):
                        pltpu.sync_copy(rowb.at[pl.ds(j * 128, 128), :],
                                        acc.at[idxb.at[j]], add=True)
                return _

            lax.fori_loop(0, _ROWS_PER_W16 // 17, body, None)
            plsc.subcore_barrier()
            pltpu.sync_copy(acc.at[pl.ds(sid * _NPT, _NPT), :],
                            agg_out.at[pl.ds(sid * _NPT, _NPT), pl.ds(c0, 8)])
            if g == 6:
                pltpu.sync_copy(zeros16.at[pl.ds(sid * _NPT, _NPT), 0:8],
                                agg_out.at[pl.ds(sid * _NPT, _NPT),
                                           pl.ds(56, 8)])
            plsc.subcore_barrier()

    return


def kernel(x, batch, node_attr, edge_src, edge_dst, embed, W_up, W_proj, Wr1,
           Wr2, B, C, Wemb, Wg, Wsi_s, Wsi_v, h, mix):
    f32 = jnp.float32
    xp = jnp.pad(x, ((0, 0), (0, 5)))
    pad_e = EP - E
    src2d = jnp.concatenate(
        [edge_src, jnp.zeros((pad_e,), jnp.int32)]).reshape(ROWS_P, 128)
    dst2d = jnp.concatenate(
        [edge_dst, jnp.zeros((pad_e,), jnp.int32)]).reshape(ROWS_P, 128)
    zeros16 = jnp.zeros((N, 16), f32)
    eye3 = jnp.eye(3, dtype=f32)

    qup = jnp.zeros((8, 24), f32).at[0:3, :].set(
        jnp.kron(W_up[None, :], eye3))
    embed_pad = jnp.pad(embed, ((0, 6), (0, 0)))
    pfull = jnp.zeros((64, 8), f32).at[32:56, 0:3].set(
        jnp.kron(W_proj[:, None], eye3))

    st = _init_call(xp, qup)
    st_old = st
    xs, xd = _geom_gather(xp, src2d, dst2d)
    ef, attr = _efeat_call(xs, xd)

    z = jnp.float32(0.0)
    for i in range(L):
        dt = jnp.clip(h[i] ** 2, 1e-4, 0.1).astype(f32)
        mm = jnp.minimum(mix[i] ** 2, 1.0).astype(f32)
        par = jnp.stack([dt, mm, z, z, z, z, z, z])[None, :]
        embw = jnp.dot(embed_pad, Wemb[i], precision=HIGH)
        k24 = jnp.kron(Wsi_v[i], eye3)
        w1m = Wr2[i][:, :NS]
        w2m = Wr2[i][:, NS:NS + NV]
        w3m = Wr2[i][:, NS + NV:2 * NS + NV]
        w4m = Wr2[i][:, 2 * NS + NV:]

        gst = _state_gather(st, src2d)
        m64 = _edge_call(ef, attr, gst, Wr1[i], w1m, w2m, w3m, w4m, B[i], C[i])
        agg = _scatter_add(m64, dst2d, zeros16)
        st_new = _node_call(agg, node_attr, st, st_old, embw, Wg[i],
                            Wsi_s[i], k24, par)
        st_old, st = st, st_new

    out8 = _proj_call(st, pfull)
    return out8[:, :3]


# speedup vs baseline: 25.7564x; 2.1088x over previous
import functools
import math

import numpy as np
import jax
import jax.numpy as jnp
from jax import lax
from jax.experimental import pallas as pl
from jax.experimental.pallas import tpu as pltpu
from jax.experimental.pallas import tpu_sc as plsc

N = 100000
E = 1600000
NS = 32
NV = 8
NB = 8
L = 4
MAX_RADIUS = 5.0
INV_NEIGH = 1.0 / math.sqrt(16.0)

ROWS = E // 128
ROWS_P = 12512
EP = ROWS_P * 128
BE = 2048
BN = 4000

HIGH = jax.lax.Precision.HIGHEST
H3 = jax.lax.Precision.DEFAULT


def _dot(a, b):
    return jnp.dot(a, b, precision=HIGH, preferred_element_type=jnp.float32)


def _dotf(a, b):
    return jnp.dot(a, b, precision=H3, preferred_element_type=jnp.float32)


_GT24_NP = np.zeros((8, 24), np.float32)
for _v in range(8):
    for _c in range(3):
        _GT24_NP[_v, 3 * _v + _c] = 1.0


def _iota2(shape, dim):
    return lax.broadcasted_iota(jnp.int32, shape, dim)


def _sel_s():
    return (_iota2((64, 32), 0) == _iota2((64, 32), 1)).astype(jnp.float32)


def _sel_v():
    return (_iota2((64, 24), 0) - 32 == _iota2((64, 24), 1)).astype(jnp.float32)


def _g_mat():
    return (_iota2((24, 8), 0) // 3 == _iota2((24, 8), 1)).astype(jnp.float32)


def _gt_mat():
    return (_iota2((8, 24), 1) // 3 == _iota2((8, 24), 0)).astype(jnp.float32)


def _t_mat():
    i0 = _iota2((8, 24), 0)
    return ((_iota2((8, 24), 1) % 3 == i0) & (i0 < 3)).astype(jnp.float32)


def _t64_mat():
    i0 = _iota2((8, 64), 0)
    j = _iota2((8, 64), 1)
    return ((j >= 32) & (j < 56) & ((j - 32) % 3 == i0)
            & (i0 < 3)).astype(jnp.float32)


def _g64_mat():
    i0 = _iota2((64, 8), 0)
    j = _iota2((64, 8), 1)
    return ((i0 >= 32) & (i0 < 56)
            & ((i0 - 32) // 3 == j)).astype(jnp.float32)


def _init_body(xp_ref, q_ref, st_ref):
    v0 = _dot(xp_ref[...], q_ref[...])
    z32 = jnp.zeros((xp_ref.shape[0], 32), jnp.float32)
    z8 = jnp.zeros((xp_ref.shape[0], 8), jnp.float32)
    st_ref[...] = jnp.concatenate([z32, v0, z8], axis=1)


def _efeat_body(xs_ref, xd_ref, ef_ref, at_ref):
    vec = xs_ref[...] - xd_ref[...]
    l2 = jnp.sum(vec * vec, axis=1, keepdims=True)
    ln = jnp.sqrt(l2)
    safe = jnp.maximum(ln, 1e-6)
    n_pi = (lax.broadcasted_iota(jnp.int32, vec.shape, 1).astype(jnp.float32)
            + 1.0) * math.pi
    ef = math.sqrt(2.0 / MAX_RADIUS) * jnp.sin(n_pi * (safe / MAX_RADIUS)) / safe
    ef_ref[...] = ef * (NB ** 0.5)
    u = 2.0 * (ln / MAX_RADIUS - 1.0)
    y = (1.0 - jnp.cos(math.pi * u)) / 2.0
    y = jnp.where(u > 0, 0.0, y)
    y = jnp.where(u < -2, 1.0, y)
    at_ref[...] = y * math.sqrt(3.0) * vec / safe


def _edge_body(ef_ref, at_ref, gst_ref, wr1_ref, wa_ref, w3p_ref, w2m_ref,
               b64_ref, cg64_ref, m_ref):
    ef = ef_ref[...]
    attr = at_ref[...]
    gst = gst_ref[...]
    hmid = jax.nn.silu(_dotf(ef, wr1_ref[...]))
    wa = _dotf(hmid, wa_ref[...])
    w3 = _dotf(hmid, w3p_ref[...])
    w2 = _dotf(hmid, w2m_ref[...])
    a64 = _dotf(attr, _t64_mat())
    dotv = _dotf(gst * a64, _g64_mat())
    m_ref[...] = (gst * wa + _dotf(dotv * w2, b64_ref[...])
                  + a64 * _dotf(gst * w3, cg64_ref[...]))


def _node_body(agg_ref, na_ref, st_ref, sto_ref, embw_ref, wg_ref, wsis_ref,
               k24_ref, par_ref, out_ref):
    agg = agg_ref[...]
    st = st_ref[...]
    sto = sto_ref[...]
    dt = par_ref[0:1, 0:1]
    mm = par_ref[0:1, 1:2]
    sel_s = _sel_s()
    sel_v = _sel_v()
    oh = (na_ref[...] == lax.broadcasted_iota(jnp.int32, (agg.shape[0], 16), 1)
          ).astype(jnp.float32)
    agg_s = _dot(agg, sel_s) * INV_NEIGH + _dot(oh, embw_ref[...])
    agg_v = _dot(agg, sel_v) * INV_NEIGH
    s = _dot(st, sel_s)
    v = _dot(st, sel_v)
    s_o = _dot(sto, sel_s)
    v_o = _dot(sto, sel_v)
    g8 = jax.nn.sigmoid(_dot(agg_s, wg_ref[...]))
    cs = jax.nn.silu(agg_s)
    cv = agg_v * _dot(g8, _gt_mat())
    si_s = _dot(s, wsis_ref[...])
    si_v = _dot(v, k24_ref[...])
    y_s = mm * cs + (1.0 - mm) * si_s
    y_v = mm * cv + (1.0 - mm) * si_v
    s_n = 2.0 * s - s_o + dt * y_s
    v_n = 2.0 * v - v_o + dt * y_v
    z8 = jnp.zeros((agg.shape[0], 8), jnp.float32)
    out_ref[...] = jnp.concatenate([s_n, v_n, z8], axis=1)


def _proj_body(st_ref, p_ref, out_ref):
    out_ref[...] = _dot(st_ref[...], p_ref[...])


def _full(shape):
    return pl.BlockSpec(shape, lambda i: tuple(0 for _ in shape))


_GE = EP // BE
_GN = N // BN

_init_call = pl.pallas_call(
    _init_body,
    grid=(_GN,),
    in_specs=[pl.BlockSpec((BN, 8), lambda i: (i, 0)), _full((8, 24))],
    out_specs=pl.BlockSpec((BN, 64), lambda i: (i, 0)),
    out_shape=jax.ShapeDtypeStruct((N, 64), jnp.float32),
)

_efeat_call = pl.pallas_call(
    _efeat_body,
    grid=(_GE,),
    in_specs=[pl.BlockSpec((BE, 8), lambda i: (i, 0))] * 2,
    out_specs=[pl.BlockSpec((BE, 8), lambda i: (i, 0))] * 2,
    out_shape=[jax.ShapeDtypeStruct((EP, 8), jnp.float32)] * 2,
)

_edge_call = pl.pallas_call(
    _edge_body,
    grid=(_GE,),
    in_specs=[
        pl.BlockSpec((BE, 8), lambda i: (i, 0)),
        pl.BlockSpec((BE, 8), lambda i: (i, 0)),
        pl.BlockSpec((BE, 64), lambda i: (i, 0)),
        _full((8, 16)), _full((16, 64)), _full((16, 64)), _full((16, 8)),
        _full((8, 64)), _full((64, 64)),
    ],
    out_specs=pl.BlockSpec((BE, 64), lambda i: (i, 0)),
    out_shape=jax.ShapeDtypeStruct((EP, 64), jnp.float32),
)

_node_call = pl.pallas_call(
    _node_body,
    grid=(_GN,),
    in_specs=[
        pl.BlockSpec((BN, 64), lambda i: (i, 0)),
        pl.BlockSpec((BN, 1), lambda i: (i, 0)),
        pl.BlockSpec((BN, 64), lambda i: (i, 0)),
        pl.BlockSpec((BN, 64), lambda i: (i, 0)),
        _full((16, 32)), _full((32, 8)), _full((32, 32)), _full((24, 24)),
        _full((1, 8)),
    ],
    out_specs=pl.BlockSpec((BN, 64), lambda i: (i, 0)),
    out_shape=jax.ShapeDtypeStruct((N, 64), jnp.float32),
)

_proj_call = pl.pallas_call(
    _proj_body,
    grid=(_GN,),
    in_specs=[pl.BlockSpec((BN, 64), lambda i: (i, 0)), _full((64, 8))],
    out_specs=pl.BlockSpec((BN, 8), lambda i: (i, 0)),
    out_shape=jax.ShapeDtypeStruct((N, 8), jnp.float32),
)


_mesh = plsc.VectorSubcoreMesh(core_axis_name="c", subcore_axis_name="s")

_ROWS_PER_W32 = ROWS_P // 32
_ROWS_PER_W16 = ROWS_P // 16
_NPT = N // 16


@functools.partial(
    pl.kernel,
    compiler_params=pltpu.CompilerParams(use_tc_tiling_on_sc=False),
    out_type=(jax.ShapeDtypeStruct((EP, 8), jnp.float32),
              jax.ShapeDtypeStruct((EP, 8), jnp.float32)),
    mesh=_mesh,
    scratch_types=[
        pltpu.VMEM((128,), jnp.int32),
        pltpu.VMEM((128,), jnp.int32),
        pltpu.VMEM((128, 8), jnp.float32),
        pltpu.VMEM((128, 8), jnp.float32),
        pltpu.SemaphoreType.DMA,
        pltpu.SemaphoreType.DMA,
    ],
)
def _geom_gather(xp_hbm, src2d, dst2d, xs_out, xd_out,
                 idxs, idxd, bufs, bufd, sems, semd):
    wid = lax.axis_index("s") * 2 + lax.axis_index("c")
    row0 = wid * _ROWS_PER_W32

    def body(j, _):
        r = row0 + j
        pltpu.sync_copy(src2d.at[r], idxs)
        pltpu.sync_copy(dst2d.at[r], idxd)
        a = pltpu.async_copy(xp_hbm.at[idxs], bufs, sems)
        b = pltpu.async_copy(xp_hbm.at[idxd], bufd, semd)
        a.wait()
        b.wait()
        pltpu.sync_copy(bufs, xs_out.at[pl.ds(r * 128, 128), :])
        pltpu.sync_copy(bufd, xd_out.at[pl.ds(r * 128, 128), :])
        return _

    lax.fori_loop(0, _ROWS_PER_W32, body, None)


@functools.partial(
    pl.kernel,
    compiler_params=pltpu.CompilerParams(use_tc_tiling_on_sc=False),
    out_type=jax.ShapeDtypeStruct((EP, 64), jnp.float32),
    mesh=_mesh,
    scratch_types=[
        pltpu.VMEM((128,), jnp.int32),
        pltpu.VMEM((128, 64), jnp.float32),
        pltpu.SemaphoreType.DMA,
    ],
)
def _state_gather(st_hbm, src2d, gst_out, idxb, buf, sem):
    wid = lax.axis_index("s") * 2 + lax.axis_index("c")
    row0 = wid * _ROWS_PER_W32

    def body(j, _):
        r = row0 + j
        pltpu.sync_copy(src2d.at[r], idxb)
        pltpu.async_copy(st_hbm.at[idxb], buf, sem).wait()
        pltpu.sync_copy(buf, gst_out.at[pl.ds(r * 128, 128), :])
        return _

    lax.fori_loop(0, _ROWS_PER_W32, body, None)


@functools.partial(
    pl.kernel,
    compiler_params=pltpu.CompilerParams(use_tc_tiling_on_sc=False),
    out_type=jax.ShapeDtypeStruct((N, 64), jnp.float32),
    mesh=_mesh,
    scratch_types=[
        pltpu.VMEM((17, 128), jnp.int32),
        pltpu.VMEM((17 * 128, 8), jnp.float32),
        pltpu.VMEM_SHARED((N, 8), jnp.float32),
    ],
)
def _scatter_add(m64, dst2d, zeros16, agg_out, idxb, rowb, acc):
    core = lax.axis_index("c")
    sid = lax.axis_index("s")
    row0 = sid * _ROWS_PER_W16

    for g in range(7):
        @pl.when(core == g // 4)
        def _group(g=g):
            c0 = 8 * g
            pltpu.sync_copy(zeros16.at[pl.ds(sid * _NPT, _NPT), 0:8],
                            acc.at[pl.ds(sid * _NPT, _NPT), :])
            plsc.subcore_barrier()

            def body(b, _):
                rb = row0 + b * 17
                pltpu.sync_copy(dst2d.at[pl.ds(rb, 17), :], idxb)
                pltpu.sync_copy(m64.at[pl.ds(rb * 128, 17 * 128),
                                       pl.ds(c0, 8)], rowb)
                for j in range(17):
                    @pl.when(rb + j < ROWS)
                    def _add(j=j):
                        pltpu.sync_copy(rowb.at[pl.ds(j * 128, 128), :],
                                        acc.at[idxb.at[j]], add=True)
                return _

            lax.fori_loop(0, _ROWS_PER_W16 // 17, body, None)
            plsc.subcore_barrier()
            pltpu.sync_copy(acc.at[pl.ds(sid * _NPT, _NPT), :],
                            agg_out.at[pl.ds(sid * _NPT, _NPT), pl.ds(c0, 8)])
            if g == 6:
                pltpu.sync_copy(zeros16.at[pl.ds(sid * _NPT, _NPT), 0:8],
                                agg_out.at[pl.ds(sid * _NPT, _NPT),
                                           pl.ds(56, 8)])
            plsc.subcore_barrier()

    return


def kernel(x, batch, node_attr, edge_src, edge_dst, embed, W_up, W_proj, Wr1,
           Wr2, B, C, Wemb, Wg, Wsi_s, Wsi_v, h, mix):
    f32 = jnp.float32
    xp = jnp.pad(x, ((0, 0), (0, 5)))
    pad_e = EP - E
    src2d = jnp.concatenate(
        [edge_src, jnp.zeros((pad_e,), jnp.int32)]).reshape(ROWS_P, 128)
    dst2d = jnp.concatenate(
        [edge_dst, jnp.zeros((pad_e,), jnp.int32)]).reshape(ROWS_P, 128)
    zeros16 = jnp.zeros((N, 16), f32)
    eye3 = jnp.eye(3, dtype=f32)

    qup = jnp.zeros((8, 24), f32).at[0:3, :].set(
        jnp.kron(W_up[None, :], eye3))
    embed_pad = jnp.pad(embed, ((0, 6), (0, 0)))
    pfull = jnp.zeros((64, 8), f32).at[32:56, 0:3].set(
        jnp.kron(W_proj[:, None], eye3))

    st = _init_call(xp, qup)
    st_old = st
    xs, xd = _geom_gather(xp, src2d, dst2d)
    ef, attr = _efeat_call(xs, xd)

    z = jnp.float32(0.0)
    for i in range(L):
        dt = jnp.clip(h[i] ** 2, 1e-4, 0.1).astype(f32)
        mm = jnp.minimum(mix[i] ** 2, 1.0).astype(f32)
        par = jnp.stack([dt, mm, z, z, z, z, z, z])[None, :]
        embw = jnp.dot(embed_pad, Wemb[i], precision=HIGH)
        k24 = jnp.kron(Wsi_v[i], eye3)
        gt24 = jnp.asarray(_GT24_NP)
        w1m = Wr2[i][:, :NS]
        w2m = Wr2[i][:, NS:NS + NV]
        w3m = Wr2[i][:, NS + NV:2 * NS + NV]
        w4m = Wr2[i][:, 2 * NS + NV:]
        wa64 = jnp.concatenate(
            [w1m, jnp.dot(w4m, gt24, precision=HIGH), jnp.zeros((16, 8), f32)],
            axis=1)
        w3p = jnp.concatenate([w3m, jnp.zeros((16, 32), f32)], axis=1)
        b64 = jnp.concatenate([B[i], jnp.zeros((8, 32), f32)], axis=1)
        cg64 = jnp.zeros((64, 64), f32).at[0:32, 32:56].set(
            jnp.dot(C[i], gt24, precision=HIGH))

        gst = _state_gather(st, src2d)
        m64 = _edge_call(ef, attr, gst, Wr1[i], wa64, w3p, w2m, b64, cg64)
        agg = _scatter_add(m64, dst2d, zeros16)
        st_new = _node_call(agg, node_attr, st, st_old, embw, Wg[i],
                            Wsi_s[i], k24, par)
        st_old, st = st, st_new

    out8 = _proj_call(st, pfull)
    return out8[:, :3]
